# baseline (speedup 1.0000x reference)
import jax
import jax.numpy as jnp
from jax import lax
from jax.experimental import pallas as pl
from jax.experimental.pallas import tpu as pltpu
from jax.experimental.pallas import tpu_sc as plsc

EMBED_DIM = 64
NUM_CORES = 2
NUM_SUBCORES = 16
NW = NUM_CORES * NUM_SUBCORES
CHUNK = 800


def _emb_body(idx_hbm, table_hbm, out_hbm, idx_v, rows_v, sem):
    wid = lax.axis_index("s") * NUM_CORES + lax.axis_index("c")
    n = idx_hbm.shape[0]
    b_per_w = n // NW
    base = wid * b_per_w
    nchunks = b_per_w // CHUNK

    def body(g, carry):
        off = base + g * CHUNK
        pltpu.sync_copy(idx_hbm.at[pl.ds(off, CHUNK)], idx_v)
        pltpu.async_copy(table_hbm.at[idx_v], rows_v, sem).wait()
        pltpu.sync_copy(rows_v, out_hbm.at[pl.ds(off, CHUNK)])
        return carry

    lax.fori_loop(0, nchunks, body, 0)


def kernel(input, table):
    batch, hist = input.shape
    flat_idx = input.reshape(-1).astype(jnp.int32)
    n = flat_idx.shape[0]
    mesh = plsc.VectorSubcoreMesh(core_axis_name="c", subcore_axis_name="s")
    out = pl.kernel(
        _emb_body,
        mesh=mesh,
        compiler_params=pltpu.CompilerParams(use_tc_tiling_on_sc=False),
        out_type=jax.ShapeDtypeStruct((n, EMBED_DIM), jnp.float32),
        scratch_types=[
            pltpu.VMEM((CHUNK,), jnp.int32),
            pltpu.VMEM((CHUNK, EMBED_DIM), jnp.float32),
            pltpu.SemaphoreType.DMA,
        ],
    )(flat_idx, table)
    return out.reshape(batch, hist, EMBED_DIM)


# speedup vs baseline: 1.0293x; 1.0293x over previous
import jax
import jax.numpy as jnp
from jax import lax
from jax.experimental import pallas as pl
from jax.experimental.pallas import tpu as pltpu
from jax.experimental.pallas import tpu_sc as plsc

EMBED_DIM = 64
NUM_CORES = 2
NUM_SUBCORES = 16
NW = NUM_CORES * NUM_SUBCORES
CHUNK = 640
NBUF = 2


def _emb_body(idx_hbm, table_hbm, out_hbm, idx_all, rows0, rows1,
              gsem0, gsem1, osem0, osem1):
    rows = (rows0, rows1)
    gsem = (gsem0, gsem1)
    osem = (osem0, osem1)
    wid = lax.axis_index("s") * NUM_CORES + lax.axis_index("c")
    nchunks = idx_hbm.shape[1]
    base = wid * (nchunks * CHUNK)

    pltpu.sync_copy(idx_hbm.at[wid], idx_all)

    for b in range(NBUF):
        pltpu.async_copy(table_hbm.at[idx_all.at[b]], rows[b], gsem[b])

    def body(g, carry):
        for b in range(NBUF):
            c = g * NBUF + b
            off = base + c * CHUNK
            pltpu.make_async_copy(
                table_hbm.at[idx_all.at[c]], rows[b], gsem[b]).wait()
            pltpu.async_copy(rows[b], out_hbm.at[pl.ds(off, CHUNK)], osem[b])

            @pl.when(c + NBUF < nchunks)
            def _():
                pltpu.make_async_copy(
                    rows[b], out_hbm.at[pl.ds(off, CHUNK)], osem[b]).wait()
                pltpu.async_copy(
                    table_hbm.at[idx_all.at[c + NBUF]], rows[b], gsem[b])
        return carry

    lax.fori_loop(0, nchunks // NBUF, body, 0)

    for b in range(NBUF):
        c = nchunks - NBUF + b
        off = base + c * CHUNK
        pltpu.make_async_copy(
            rows[b], out_hbm.at[pl.ds(off, CHUNK)], osem[b]).wait()


def kernel(input, table):
    batch, hist = input.shape
    flat_idx = input.reshape(-1).astype(jnp.int32)
    n = flat_idx.shape[0]
    b_per_w = n // NW
    nchunks = b_per_w // CHUNK
    idx3 = flat_idx.reshape(NW, nchunks, CHUNK)
    mesh = plsc.VectorSubcoreMesh(core_axis_name="c", subcore_axis_name="s")
    out = pl.kernel(
        _emb_body,
        mesh=mesh,
        compiler_params=pltpu.CompilerParams(use_tc_tiling_on_sc=False),
        out_type=jax.ShapeDtypeStruct((n, EMBED_DIM), jnp.float32),
        scratch_types=[
            pltpu.VMEM((nchunks, CHUNK), jnp.int32),
            pltpu.VMEM((CHUNK, EMBED_DIM), jnp.float32),
            pltpu.VMEM((CHUNK, EMBED_DIM), jnp.float32),
            pltpu.SemaphoreType.DMA,
            pltpu.SemaphoreType.DMA,
            pltpu.SemaphoreType.DMA,
            pltpu.SemaphoreType.DMA,
        ],
    )(idx3, table)
    return out.reshape(batch, hist, EMBED_DIM)
